# baseline (speedup 1.0000x reference)
import jax
import jax.numpy as jnp
from jax import lax
from jax.experimental import pallas as pl
from jax.experimental.pallas import tpu as pltpu
from jax.experimental.pallas import tpu_sc as plsc

MAXLEN = 200
EMBED = 128
BATCH = 1024
NW = 32
BPW = BATCH // NW
HALF = MAXLEN // 2
LANES = 16
NBUF = 3


def _body(x_hbm, tok_hbm, pos_hbm, out_hbm, pos_v, idx_v, rows0, rows1, rows2,
          sg0, sg1, sg2, so0, so1, so2):
    wid = lax.axis_index("s") * 2 + lax.axis_index("c")
    pltpu.sync_copy(pos_hbm, pos_v)
    pltpu.sync_copy(x_hbm.at[wid], idx_v)

    rows = (rows0, rows1, rows2)
    sg = (sg0, sg1, sg2)
    so = (so0, so1, so2)
    store_desc = [None, None, None]
    gather_desc = [None, None, None]

    def start_gather(i):
        b = i % NBUF
        gather_desc[b] = (
            pltpu.async_copy(
                tok_hbm.at[idx_v.at[2 * i]], rows[b].at[pl.ds(0, HALF)], sg[b]
            ),
            pltpu.async_copy(
                tok_hbm.at[idx_v.at[2 * i + 1]],
                rows[b].at[pl.ds(HALF, HALF)],
                sg[b],
            ),
        )

    start_gather(0)
    start_gather(1)
    for i in range(BPW):
        b = i % NBUF
        if i + 2 < BPW:
            nb = (i + 2) % NBUF
            if store_desc[nb] is not None:
                store_desc[nb].wait()
                store_desc[nb] = None
            start_gather(i + 2)
        gather_desc[b][0].wait()
        gather_desc[b][1].wait()

        @pl.loop(0, MAXLEN, unroll=2)
        def _row(r):
            slices = [pl.ds(c * LANES, LANES) for c in range(EMBED // LANES)]
            pv = [pos_v[r, sl] for sl in slices]
            tv = [rows[b][r, sl] for sl in slices]
            for c, sl in enumerate(slices):
                rows[b][r, sl] = tv[c] + pv[c]

        store_desc[b] = pltpu.async_copy(rows[b], out_hbm.at[wid * BPW + i], so[b])

    for d in store_desc:
        if d is not None:
            d.wait()


def kernel(x, token_table, pos_table):
    x3 = x.reshape(NW, 2 * BPW, HALF).astype(jnp.int32)
    mesh = plsc.VectorSubcoreMesh(core_axis_name="c", subcore_axis_name="s")
    f = pl.kernel(
        _body,
        out_type=jax.ShapeDtypeStruct((BATCH, MAXLEN, EMBED), jnp.float32),
        mesh=mesh,
        scratch_types=[
            pltpu.VMEM((MAXLEN, EMBED), jnp.float32),
            pltpu.VMEM((2 * BPW, HALF), jnp.int32),
            pltpu.VMEM((MAXLEN, EMBED), jnp.float32),
            pltpu.VMEM((MAXLEN, EMBED), jnp.float32),
            pltpu.VMEM((MAXLEN, EMBED), jnp.float32),
            pltpu.SemaphoreType.DMA,
            pltpu.SemaphoreType.DMA,
            pltpu.SemaphoreType.DMA,
            pltpu.SemaphoreType.DMA,
            pltpu.SemaphoreType.DMA,
            pltpu.SemaphoreType.DMA,
        ],
    )
    return f(x3, token_table, pos_table)


# speedup vs baseline: 1.2217x; 1.2217x over previous
import jax
import jax.numpy as jnp
from jax import lax
from jax.experimental import pallas as pl
from jax.experimental.pallas import tpu as pltpu
from jax.experimental.pallas import tpu_sc as plsc

MAXLEN = 200
EMBED = 128
BATCH = 1024
NW = 32
BPW = BATCH // NW
TG = 4
NCH = MAXLEN // TG
ROWS = TG * BPW
LANES = 16
NBUF = 4
MAIN = NCH - 2


def _body(x_hbm, tok_hbm, pos_hbm, out_hbm, pos_v, idx_v, bufs, sgs, sos):
    wid = lax.axis_index("s") * 2 + lax.axis_index("c")
    base = wid * BPW
    pltpu.sync_copy(pos_hbm, pos_v)
    pltpu.sync_copy(x_hbm.at[wid], idx_v)

    def gather(k, b):
        pltpu.async_copy(tok_hbm.at[idx_v.at[k]], bufs[b], sgs[b])

    def wait_gather(b):
        pltpu.make_async_copy(tok_hbm.at[pl.ds(0, ROWS)], bufs[b], sgs[b]).wait()

    def wait_store(b):
        pltpu.make_async_copy(
            bufs[b], out_hbm.at[0, pl.ds(0, ROWS)], sos[b]
        ).wait()

    def add_and_store(k, b):
        slices = [pl.ds(c * LANES, LANES) for c in range(EMBED // LANES)]
        pv = [
            [pos_v[TG * k + tt, sl] for sl in slices] for tt in range(TG)
        ]

        @pl.loop(0, BPW)
        def _j(j):
            for tt in range(TG):
                r = tt * BPW + j
                for c, sl in enumerate(slices):
                    bufs[b][r, sl] = bufs[b][r, sl] + pv[tt][c]

        for tt in range(TG):
            pltpu.async_copy(
                bufs[b].at[pl.ds(tt * BPW, BPW)],
                out_hbm.at[pl.ds(base, BPW), TG * k + tt],
                sos[b],
            )

    gather(0, 0)
    gather(1, 1)

    @pl.loop(0, MAIN // NBUF)
    def _p(p):
        for bb in range(NBUF):
            k = NBUF * p + bb
            nb = (bb + 2) % NBUF
            if bb < 2:
                @pl.when(p > 0)
                def _w():
                    wait_store(nb)
            else:
                wait_store(nb)
            gather(k + 2, nb)
            wait_gather(bb)
            add_and_store(k, bb)

    for k in (MAIN, MAIN + 1):
        b = k % NBUF
        wait_gather(b)
        add_and_store(k, b)
    for b in range(NBUF):
        wait_store(b)


def _kernel_body(x_hbm, tok_hbm, pos_hbm, out_hbm, pos_v, idx_v,
                 buf0, buf1, buf2, buf3, sg0, sg1, sg2, sg3,
                 so0, so1, so2, so3):
    _body(x_hbm, tok_hbm, pos_hbm, out_hbm, pos_v, idx_v,
          (buf0, buf1, buf2, buf3), (sg0, sg1, sg2, sg3),
          (so0, so1, so2, so3))


def kernel(x, token_table, pos_table):
    xt = (
        x.reshape(NW, BPW, NCH, TG)
        .transpose(0, 2, 3, 1)
        .reshape(NW, NCH, ROWS)
        .astype(jnp.int32)
    )
    mesh = plsc.VectorSubcoreMesh(core_axis_name="c", subcore_axis_name="s")
    f = pl.kernel(
        _kernel_body,
        out_type=jax.ShapeDtypeStruct((BATCH, MAXLEN, EMBED), jnp.float32),
        mesh=mesh,
        scratch_types=[
            pltpu.VMEM((MAXLEN, EMBED), jnp.float32),
            pltpu.VMEM((NCH, ROWS), jnp.int32),
            pltpu.VMEM((ROWS, EMBED), jnp.float32),
            pltpu.VMEM((ROWS, EMBED), jnp.float32),
            pltpu.VMEM((ROWS, EMBED), jnp.float32),
            pltpu.VMEM((ROWS, EMBED), jnp.float32),
            pltpu.SemaphoreType.DMA,
            pltpu.SemaphoreType.DMA,
            pltpu.SemaphoreType.DMA,
            pltpu.SemaphoreType.DMA,
            pltpu.SemaphoreType.DMA,
            pltpu.SemaphoreType.DMA,
            pltpu.SemaphoreType.DMA,
            pltpu.SemaphoreType.DMA,
        ],
    )
    return f(xt, token_table, pos_table)
